# baseline (speedup 1.0000x reference)
import functools

import jax
import jax.numpy as jnp
from jax import lax
from jax.experimental import pallas as pl
from jax.experimental.pallas import tpu as pltpu
from jax.experimental.pallas import tpu_sc as plsc

D_MODEL = 768
D_FF = 3072
NUM_EXPERTS = 8
TOP_K = 2
SEQ = 2048
ASSIGN = SEQ * TOP_K

TOK_BLK = 512
ROW_BLK = 128
NUM_BLOCKS = ASSIGN // ROW_BLK + NUM_EXPERTS
NPAD = NUM_BLOCKS * ROW_BLK
D_HALF = D_MODEL // 2


def _rne_bf16(u):
    return u + jnp.uint32(0x7FFF) + ((u >> 16) & jnp.uint32(1))


def _pack_halves(f):
    u = lax.bitcast_convert_type(f, jnp.uint32)
    left = _rne_bf16(u[:, :D_HALF]) & jnp.uint32(0xFFFF0000)
    right = _rne_bf16(u[:, D_HALF:]) >> 16
    return left | right


def _unpack_halves(p):
    lf = lax.bitcast_convert_type(p & jnp.uint32(0xFFFF0000), jnp.float32)
    rf = lax.bitcast_convert_type(p << 16, jnp.float32)
    return jnp.concatenate([lf, rf], axis=1)


def _gate_body(x_ref, wg_ref, sim_ref, temp_ref, probs_ref, eidx_ref,
               rank_ref, counts_ref, xpk_ref, cnt_scratch):
    pid = pl.program_id(0)

    @pl.when(pid == 0)
    def _():
        cnt_scratch[...] = jnp.zeros_like(cnt_scratch)

    xb = x_ref[...]
    xpk_ref[...] = _pack_halves(xb)
    proj = lax.dot_general(xb, wg_ref[...],
                           (((1,), (1,)), ((), ())),
                           preferred_element_type=jnp.float32)
    pn = proj / jnp.maximum(
        jnp.sqrt(jnp.sum(proj * proj, axis=1, keepdims=True)), 1e-12)
    sim = sim_ref[...]
    simn = sim / jnp.maximum(
        jnp.sqrt(jnp.sum(sim * sim, axis=1, keepdims=True)), 1e-12)
    scores = lax.dot_general(pn, simn, (((1,), (1,)), ((), ())),
                             preferred_element_type=jnp.float32)
    scores = scores / temp_ref[0, 0]

    ecol = lax.broadcasted_iota(jnp.int32, (TOK_BLK, NUM_EXPERTS), 1)
    s0 = jnp.max(scores, axis=1, keepdims=True)
    i0 = jnp.min(jnp.where(scores == s0, ecol, NUM_EXPERTS), axis=1,
                 keepdims=True)
    masked = jnp.where(ecol == i0, -jnp.inf, scores)
    s1 = jnp.max(masked, axis=1, keepdims=True)
    i1 = jnp.min(jnp.where(masked == s1, ecol, NUM_EXPERTS), axis=1,
                 keepdims=True)

    e1 = jnp.exp(s1 - s0)
    p0 = 1.0 / (1.0 + e1)
    p1 = e1 * p0
    probs_ref[...] = jnp.concatenate([p0, p1], axis=1)
    eidx_ref[...] = jnp.concatenate([i0, i1], axis=1)

    h0 = (ecol == i0).astype(jnp.float32)
    h1 = (ecol == i1).astype(jnp.float32)
    h = jnp.concatenate([h0, h1], axis=0)
    n2 = 2 * TOK_BLK
    rr = lax.broadcasted_iota(jnp.int32, (n2, n2), 0)
    cc = lax.broadcasted_iota(jnp.int32, (n2, n2), 1)
    lstrict = (cc < rr).astype(jnp.float32)
    cum = lax.dot_general(lstrict, h, (((1,), (0,)), ((), ())),
                          preferred_element_type=jnp.float32)
    carry = cnt_scratch[...].astype(jnp.float32)
    rank_local = jnp.sum(h * cum, axis=1, keepdims=True)
    rank_carry = jnp.sum(h * carry, axis=1, keepdims=True)
    rank = (rank_local + rank_carry).astype(jnp.int32)
    rank_ref[...] = jnp.concatenate(
        [rank[:TOK_BLK], rank[TOK_BLK:]], axis=1)

    new_cnt = cnt_scratch[...] + jnp.sum(h, axis=0,
                                         keepdims=True).astype(jnp.int32)
    cnt_scratch[...] = new_cnt
    counts_ref[...] = new_cnt


def _gate(x2d, wg, sim, temp):
    nblk = SEQ // TOK_BLK
    return pl.pallas_call(
        _gate_body,
        grid=(nblk,),
        in_specs=[
            pl.BlockSpec((TOK_BLK, D_MODEL), lambda i: (i, 0)),
            pl.BlockSpec((D_MODEL, D_MODEL), lambda i: (0, 0)),
            pl.BlockSpec((NUM_EXPERTS, D_MODEL), lambda i: (0, 0)),
            pl.BlockSpec(memory_space=pltpu.SMEM),
        ],
        out_specs=[
            pl.BlockSpec((TOK_BLK, TOP_K), lambda i: (i, 0)),
            pl.BlockSpec((TOK_BLK, TOP_K), lambda i: (i, 0)),
            pl.BlockSpec((TOK_BLK, TOP_K), lambda i: (i, 0)),
            pl.BlockSpec((1, NUM_EXPERTS), lambda i: (0, 0)),
            pl.BlockSpec((TOK_BLK, D_HALF), lambda i: (i, 0)),
        ],
        out_shape=[
            jax.ShapeDtypeStruct((SEQ, TOP_K), jnp.float32),
            jax.ShapeDtypeStruct((SEQ, TOP_K), jnp.int32),
            jax.ShapeDtypeStruct((SEQ, TOP_K), jnp.int32),
            jax.ShapeDtypeStruct((1, NUM_EXPERTS), jnp.int32),
            jax.ShapeDtypeStruct((SEQ, D_HALF), jnp.uint32),
        ],
        scratch_shapes=[pltpu.VMEM((1, NUM_EXPERTS), jnp.int32)],
    )(x2d, wg, sim, temp)


def _mlp_body(scal_ref, x_ref, w1_hbm, b1_ref, w2_hbm, b2_ref, y_ref,
              w1b, w2b, sems):
    nb = NUM_BLOCKS
    g = pl.program_id(0)
    used = scal_ref[4 * nb]
    be_g = scal_ref[g]
    chg = scal_ref[nb + g]
    slot = scal_ref[2 * nb + g]
    nxte = scal_ref[3 * nb + g]

    @pl.when(g == 0)
    def _():
        pltpu.make_async_copy(w1_hbm.at[be_g], w1b.at[0],
                              sems.at[0, 0]).start()
        pltpu.make_async_copy(w2_hbm.at[be_g], w2b.at[0],
                              sems.at[0, 1]).start()

    @pl.when(chg == 1)
    def _():
        pltpu.make_async_copy(w1_hbm.at[be_g], w1b.at[slot],
                              sems.at[slot, 0]).wait()
        pltpu.make_async_copy(w2_hbm.at[be_g], w2b.at[slot],
                              sems.at[slot, 1]).wait()

        @pl.when(nxte >= 0)
        def _():
            pltpu.make_async_copy(w1_hbm.at[nxte], w1b.at[1 - slot],
                                  sems.at[1 - slot, 0]).start()
            pltpu.make_async_copy(w2_hbm.at[nxte], w2b.at[1 - slot],
                                  sems.at[1 - slot, 1]).start()

    @pl.when(g < used)
    def _():
        xb = _unpack_halves(x_ref[...]).astype(jnp.bfloat16)
        h = lax.dot_general(xb, w1b[slot].astype(jnp.bfloat16),
                            (((1,), (1,)), ((), ())),
                            preferred_element_type=jnp.float32)
        h = h + b1_ref[0]
        h = 0.5 * h * (1.0 + lax.erf(h * 0.7071067811865476))
        y = lax.dot_general(h.astype(jnp.bfloat16),
                            w2b[slot].astype(jnp.bfloat16),
                            (((1,), (1,)), ((), ())),
                            preferred_element_type=jnp.float32)
        y_ref[...] = _pack_halves(y + b2_ref[0])


def _grouped_mlp(x_sorted, scal, w1, b1, w2, b2):
    grid_spec = pltpu.PrefetchScalarGridSpec(
        num_scalar_prefetch=1,
        grid=(NUM_BLOCKS,),
        in_specs=[
            pl.BlockSpec((ROW_BLK, D_HALF), lambda g, be: (g, 0)),
            pl.BlockSpec(memory_space=pl.ANY),
            pl.BlockSpec((1, 1, D_FF), lambda g, be: (be[g], 0, 0)),
            pl.BlockSpec(memory_space=pl.ANY),
            pl.BlockSpec((1, 1, D_MODEL), lambda g, be: (be[g], 0, 0)),
        ],
        out_specs=pl.BlockSpec((ROW_BLK, D_HALF), lambda g, be: (g, 0)),
        scratch_shapes=[
            pltpu.VMEM((2, D_FF, D_MODEL), jnp.float32),
            pltpu.VMEM((2, D_MODEL, D_FF), jnp.float32),
            pltpu.SemaphoreType.DMA((2, 2)),
        ],
    )
    return pl.pallas_call(
        _mlp_body,
        grid_spec=grid_spec,
        out_shape=jax.ShapeDtypeStruct((NPAD, D_HALF), jnp.uint32),
    )(scal, x_sorted, w1, b1.reshape(NUM_EXPERTS, 1, D_FF),
      w2, b2.reshape(NUM_EXPERTS, 1, D_MODEL))


SC_CORES = 2
SC_SUBCORES = 16
SC_WORKERS = SC_CORES * SC_SUBCORES
TOK_CHUNK = SEQ // SC_WORKERS


def _sc_scatter_body(x_hbm, pos0_hbm, pos1_hbm, out_hbm, idx0_v, idx1_v,
                     x_v, sem):
    wid = lax.axis_index("s") * SC_CORES + lax.axis_index("c")
    base = wid * TOK_CHUNK
    pltpu.sync_copy(pos0_hbm.at[pl.ds(base, TOK_CHUNK)], idx0_v)
    pltpu.sync_copy(pos1_hbm.at[pl.ds(base, TOK_CHUNK)], idx1_v)
    pltpu.sync_copy(x_hbm.at[pl.ds(base, TOK_CHUNK)], x_v)
    pltpu.async_copy(x_v, out_hbm.at[idx0_v], sem).wait()
    pltpu.async_copy(x_v, out_hbm.at[idx1_v], sem).wait()


def _sc_scatter(xpk, pos0, pos1):
    mesh = plsc.VectorSubcoreMesh(core_axis_name="c", subcore_axis_name="s")
    f = pl.kernel(
        _sc_scatter_body,
        out_type=jax.ShapeDtypeStruct((NPAD, D_HALF), jnp.uint32),
        mesh=mesh,
        scratch_types=[
            pltpu.VMEM((TOK_CHUNK,), jnp.int32),
            pltpu.VMEM((TOK_CHUNK,), jnp.int32),
            pltpu.VMEM((TOK_CHUNK, D_HALF), jnp.uint32),
            pltpu.SemaphoreType.DMA,
        ],
    )
    return f(xpk, pos0, pos1)


def _sc_gather_body(y_hbm, pos0_hbm, pos1_hbm, y0_hbm, y1_hbm, idx_v, r_v,
                    sem):
    wid = lax.axis_index("s") * SC_CORES + lax.axis_index("c")
    base = wid * TOK_CHUNK
    pltpu.sync_copy(pos0_hbm.at[pl.ds(base, TOK_CHUNK)], idx_v)
    pltpu.async_copy(y_hbm.at[idx_v], r_v, sem).wait()
    pltpu.sync_copy(r_v, y0_hbm.at[pl.ds(base, TOK_CHUNK)])
    pltpu.sync_copy(pos1_hbm.at[pl.ds(base, TOK_CHUNK)], idx_v)
    pltpu.async_copy(y_hbm.at[idx_v], r_v, sem).wait()
    pltpu.sync_copy(r_v, y1_hbm.at[pl.ds(base, TOK_CHUNK)])


def _sc_gather(y_sorted, pos0, pos1):
    mesh = plsc.VectorSubcoreMesh(core_axis_name="c", subcore_axis_name="s")
    f = pl.kernel(
        _sc_gather_body,
        out_type=[
            jax.ShapeDtypeStruct((SEQ, D_HALF), jnp.uint32),
            jax.ShapeDtypeStruct((SEQ, D_HALF), jnp.uint32),
        ],
        mesh=mesh,
        scratch_types=[
            pltpu.VMEM((TOK_CHUNK,), jnp.int32),
            pltpu.VMEM((TOK_CHUNK, D_HALF), jnp.uint32),
            pltpu.SemaphoreType.DMA,
        ],
    )
    return f(y_sorted, pos0, pos1)


def _combine_body(y0_ref, y1_ref, probs_ref, out_ref):
    p = probs_ref[...]
    y0 = _unpack_halves(y0_ref[...])
    y1 = _unpack_halves(y1_ref[...])
    out_ref[...] = p[:, 0:1] * y0 + p[:, 1:2] * y1


def _combine(y0, y1, probs):
    nblk = SEQ // TOK_BLK
    return pl.pallas_call(
        _combine_body,
        grid=(nblk,),
        in_specs=[
            pl.BlockSpec((TOK_BLK, D_HALF), lambda i: (i, 0)),
            pl.BlockSpec((TOK_BLK, D_HALF), lambda i: (i, 0)),
            pl.BlockSpec((TOK_BLK, TOP_K), lambda i: (i, 0)),
        ],
        out_specs=pl.BlockSpec((TOK_BLK, D_MODEL), lambda i: (i, 0)),
        out_shape=jax.ShapeDtypeStruct((SEQ, D_MODEL), jnp.float32),
    )(y0, y1, probs)


def kernel(x, Wg, sim_matrix, temperature, W1, b1, W2, b2):
    x2d = x.reshape(SEQ, D_MODEL)
    temp = temperature.reshape(1, 1)

    probs, eidx, rank, counts, xpk = _gate(x2d, Wg, sim_matrix, temp)
    counts = counts[0]

    nb = (counts + ROW_BLK - 1) // ROW_BLK
    cum_nb = jnp.cumsum(nb)
    first_blk = cum_nb - nb
    off_rows = first_blk * ROW_BLK
    erange = jnp.arange(NUM_EXPERTS, dtype=jnp.int32)
    pos = rank + jnp.sum(
        jnp.where(eidx[..., None] == erange, off_rows, 0), axis=-1)

    gids = jnp.arange(NUM_BLOCKS, dtype=jnp.int32)
    be_raw = jnp.sum(gids[:, None] >= cum_nb[None, :], axis=1)
    used = cum_nb[-1]
    last_e = jnp.clip(jnp.sum((used - 1) >= cum_nb), 0, NUM_EXPERTS - 1)
    block_expert = jnp.where(gids < used, jnp.clip(be_raw, 0, NUM_EXPERTS - 1),
                             last_e).astype(jnp.int32)

    pos0 = pos[:, 0]
    pos1 = pos[:, 1]
    x_sorted = _sc_scatter(xpk, pos0, pos1)

    prev_be = jnp.roll(block_expert, 1)
    chg = ((gids < used) & ((gids == 0) | (block_expert != prev_be)))
    eord = jnp.cumsum(chg.astype(jnp.int32)) - 1
    slot_arr = jnp.where(gids < used, eord % 2, 0).astype(jnp.int32)
    cand = jnp.where(chg, gids, NUM_BLOCKS)
    later = jnp.where(gids[None, :] > gids[:, None], cand[None, :],
                      NUM_BLOCKS)
    su = jnp.min(later, axis=1)
    nxte_val = jnp.sum(
        jnp.where(su[:, None] == gids[None, :], block_expert[None, :], 0),
        axis=1)
    nxte = jnp.where(su < NUM_BLOCKS, nxte_val, -1).astype(jnp.int32)
    scal = jnp.concatenate(
        [block_expert, chg.astype(jnp.int32), slot_arr, nxte,
         used.astype(jnp.int32).reshape(1)])
    y_sorted = _grouped_mlp(x_sorted, scal, W1, b1, W2, b2)

    y0, y1 = _sc_gather(y_sorted, pos0, pos1)
    out = _combine(y0, y1, probs)
    return out.reshape(x.shape)


# speedup vs baseline: 1.4751x; 1.4751x over previous
import functools

import jax
import jax.numpy as jnp
from jax import lax
from jax.experimental import pallas as pl
from jax.experimental.pallas import tpu as pltpu
from jax.experimental.pallas import tpu_sc as plsc

D_MODEL = 768
D_FF = 3072
NUM_EXPERTS = 8
TOP_K = 2
SEQ = 2048
ASSIGN = SEQ * TOP_K

TOK_BLK = 512
ROW_BLK = 256
NUM_BLOCKS = ASSIGN // ROW_BLK + NUM_EXPERTS
NPAD = NUM_BLOCKS * ROW_BLK
D_HALF = D_MODEL // 2


def _rne_bf16(u):
    return u + jnp.uint32(0x7FFF) + ((u >> 16) & jnp.uint32(1))


def _pack_halves(f):
    u = lax.bitcast_convert_type(f, jnp.uint32)
    left = _rne_bf16(u[:, :D_HALF]) & jnp.uint32(0xFFFF0000)
    right = _rne_bf16(u[:, D_HALF:]) >> 16
    return left | right


def _unpack_halves(p):
    lf = lax.bitcast_convert_type(p & jnp.uint32(0xFFFF0000), jnp.float32)
    rf = lax.bitcast_convert_type(p << 16, jnp.float32)
    return jnp.concatenate([lf, rf], axis=1)


def _gate_body(x_ref, wg_ref, sim_ref, temp_ref, probs_ref, eidx_ref,
               rank_ref, counts_ref, xpk_ref, cnt_scratch):
    pid = pl.program_id(0)

    @pl.when(pid == 0)
    def _():
        cnt_scratch[...] = jnp.zeros_like(cnt_scratch)

    xb = x_ref[...]
    xpk_ref[...] = _pack_halves(xb)
    proj = lax.dot_general(xb, wg_ref[...],
                           (((1,), (1,)), ((), ())),
                           preferred_element_type=jnp.float32)
    pn = proj / jnp.maximum(
        jnp.sqrt(jnp.sum(proj * proj, axis=1, keepdims=True)), 1e-12)
    sim = sim_ref[...]
    simn = sim / jnp.maximum(
        jnp.sqrt(jnp.sum(sim * sim, axis=1, keepdims=True)), 1e-12)
    scores = lax.dot_general(pn, simn, (((1,), (1,)), ((), ())),
                             preferred_element_type=jnp.float32)
    scores = scores / temp_ref[0, 0]

    ecol = lax.broadcasted_iota(jnp.int32, (TOK_BLK, NUM_EXPERTS), 1)
    s0 = jnp.max(scores, axis=1, keepdims=True)
    i0 = jnp.min(jnp.where(scores == s0, ecol, NUM_EXPERTS), axis=1,
                 keepdims=True)
    masked = jnp.where(ecol == i0, -jnp.inf, scores)
    s1 = jnp.max(masked, axis=1, keepdims=True)
    i1 = jnp.min(jnp.where(masked == s1, ecol, NUM_EXPERTS), axis=1,
                 keepdims=True)

    e1 = jnp.exp(s1 - s0)
    p0 = 1.0 / (1.0 + e1)
    p1 = e1 * p0
    probs_ref[...] = jnp.concatenate([p0, p1], axis=1)
    eidx_ref[...] = jnp.concatenate([i0, i1], axis=1)

    h0 = (ecol == i0).astype(jnp.float32)
    h1 = (ecol == i1).astype(jnp.float32)
    h = jnp.concatenate([h0, h1], axis=0)
    n2 = 2 * TOK_BLK
    rr = lax.broadcasted_iota(jnp.int32, (n2, n2), 0)
    cc = lax.broadcasted_iota(jnp.int32, (n2, n2), 1)
    lstrict = (cc < rr).astype(jnp.float32)
    cum = lax.dot_general(lstrict, h, (((1,), (0,)), ((), ())),
                          preferred_element_type=jnp.float32)
    carry = cnt_scratch[...].astype(jnp.float32)
    rank_local = jnp.sum(h * cum, axis=1, keepdims=True)
    rank_carry = jnp.sum(h * carry, axis=1, keepdims=True)
    rank = (rank_local + rank_carry).astype(jnp.int32)
    rank_ref[...] = jnp.concatenate(
        [rank[:TOK_BLK], rank[TOK_BLK:]], axis=1)

    new_cnt = cnt_scratch[...] + jnp.sum(h, axis=0,
                                         keepdims=True).astype(jnp.int32)
    cnt_scratch[...] = new_cnt
    counts_ref[...] = new_cnt


def _gate(x2d, wg, sim, temp):
    nblk = SEQ // TOK_BLK
    return pl.pallas_call(
        _gate_body,
        grid=(nblk,),
        in_specs=[
            pl.BlockSpec((TOK_BLK, D_MODEL), lambda i: (i, 0)),
            pl.BlockSpec((D_MODEL, D_MODEL), lambda i: (0, 0)),
            pl.BlockSpec((NUM_EXPERTS, D_MODEL), lambda i: (0, 0)),
            pl.BlockSpec(memory_space=pltpu.SMEM),
        ],
        out_specs=[
            pl.BlockSpec((TOK_BLK, TOP_K), lambda i: (i, 0)),
            pl.BlockSpec((TOK_BLK, TOP_K), lambda i: (i, 0)),
            pl.BlockSpec((TOK_BLK, TOP_K), lambda i: (i, 0)),
            pl.BlockSpec((1, NUM_EXPERTS), lambda i: (0, 0)),
            pl.BlockSpec((TOK_BLK, D_HALF), lambda i: (i, 0)),
        ],
        out_shape=[
            jax.ShapeDtypeStruct((SEQ, TOP_K), jnp.float32),
            jax.ShapeDtypeStruct((SEQ, TOP_K), jnp.int32),
            jax.ShapeDtypeStruct((SEQ, TOP_K), jnp.int32),
            jax.ShapeDtypeStruct((1, NUM_EXPERTS), jnp.int32),
            jax.ShapeDtypeStruct((SEQ, D_HALF), jnp.uint32),
        ],
        scratch_shapes=[pltpu.VMEM((1, NUM_EXPERTS), jnp.int32)],
    )(x2d, wg, sim, temp)


def _mlp_body(scal_ref, x_ref, w1_hbm, b1_ref, w2_hbm, b2_ref, y_ref,
              w1b, w2b, sems):
    nb = NUM_BLOCKS
    g = pl.program_id(0)
    used = scal_ref[4 * nb]
    be_g = scal_ref[g]
    chg = scal_ref[nb + g]
    slot = scal_ref[2 * nb + g]
    nxte = scal_ref[3 * nb + g]

    @pl.when(g == 0)
    def _():
        pltpu.make_async_copy(w1_hbm.at[be_g], w1b.at[0],
                              sems.at[0, 0]).start()
        pltpu.make_async_copy(w2_hbm.at[be_g], w2b.at[0],
                              sems.at[0, 1]).start()

    @pl.when(chg == 1)
    def _():
        pltpu.make_async_copy(w1_hbm.at[be_g], w1b.at[slot],
                              sems.at[slot, 0]).wait()
        pltpu.make_async_copy(w2_hbm.at[be_g], w2b.at[slot],
                              sems.at[slot, 1]).wait()

        @pl.when(nxte >= 0)
        def _():
            pltpu.make_async_copy(w1_hbm.at[nxte], w1b.at[1 - slot],
                                  sems.at[1 - slot, 0]).start()
            pltpu.make_async_copy(w2_hbm.at[nxte], w2b.at[1 - slot],
                                  sems.at[1 - slot, 1]).start()

    @pl.when(g < used)
    def _():
        xb = _unpack_halves(x_ref[...]).astype(jnp.bfloat16)
        h = lax.dot_general(xb, w1b[slot].astype(jnp.bfloat16),
                            (((1,), (1,)), ((), ())),
                            preferred_element_type=jnp.float32)
        h = h + b1_ref[0]
        h = 0.5 * h * (1.0 + lax.erf(h * 0.7071067811865476))
        y = lax.dot_general(h.astype(jnp.bfloat16),
                            w2b[slot].astype(jnp.bfloat16),
                            (((1,), (1,)), ((), ())),
                            preferred_element_type=jnp.float32)
        y_ref[...] = _pack_halves(y + b2_ref[0])


def _grouped_mlp(x_sorted, scal, w1, b1, w2, b2):
    grid_spec = pltpu.PrefetchScalarGridSpec(
        num_scalar_prefetch=1,
        grid=(NUM_BLOCKS,),
        in_specs=[
            pl.BlockSpec((ROW_BLK, D_HALF), lambda g, be: (g, 0)),
            pl.BlockSpec(memory_space=pl.ANY),
            pl.BlockSpec((1, 1, D_FF), lambda g, be: (be[g], 0, 0)),
            pl.BlockSpec(memory_space=pl.ANY),
            pl.BlockSpec((1, 1, D_MODEL), lambda g, be: (be[g], 0, 0)),
        ],
        out_specs=pl.BlockSpec((ROW_BLK, D_HALF), lambda g, be: (g, 0)),
        scratch_shapes=[
            pltpu.VMEM((2, D_FF, D_MODEL), jnp.float32),
            pltpu.VMEM((2, D_MODEL, D_FF), jnp.float32),
            pltpu.SemaphoreType.DMA((2, 2)),
        ],
    )
    return pl.pallas_call(
        _mlp_body,
        grid_spec=grid_spec,
        out_shape=jax.ShapeDtypeStruct((NPAD, D_HALF), jnp.uint32),
    )(scal, x_sorted, w1, b1.reshape(NUM_EXPERTS, 1, D_FF),
      w2, b2.reshape(NUM_EXPERTS, 1, D_MODEL))


SC_CORES = 2
SC_SUBCORES = 16
SC_WORKERS = SC_CORES * SC_SUBCORES
TOK_CHUNK = SEQ // SC_WORKERS


def _sc_scatter_body(x_hbm, pos0_hbm, pos1_hbm, out_hbm, idx0_v, idx1_v,
                     x_v, sem):
    wid = lax.axis_index("s") * SC_CORES + lax.axis_index("c")
    base = wid * TOK_CHUNK
    pltpu.sync_copy(pos0_hbm.at[pl.ds(base, TOK_CHUNK)], idx0_v)
    pltpu.sync_copy(pos1_hbm.at[pl.ds(base, TOK_CHUNK)], idx1_v)
    pltpu.sync_copy(x_hbm.at[pl.ds(base, TOK_CHUNK)], x_v)
    pltpu.async_copy(x_v, out_hbm.at[idx0_v], sem).wait()
    pltpu.async_copy(x_v, out_hbm.at[idx1_v], sem).wait()


def _sc_scatter(xpk, pos0, pos1):
    mesh = plsc.VectorSubcoreMesh(core_axis_name="c", subcore_axis_name="s")
    f = pl.kernel(
        _sc_scatter_body,
        out_type=jax.ShapeDtypeStruct((NPAD, D_HALF), jnp.uint32),
        mesh=mesh,
        scratch_types=[
            pltpu.VMEM((TOK_CHUNK,), jnp.int32),
            pltpu.VMEM((TOK_CHUNK,), jnp.int32),
            pltpu.VMEM((TOK_CHUNK, D_HALF), jnp.uint32),
            pltpu.SemaphoreType.DMA,
        ],
    )
    return f(xpk, pos0, pos1)


def _sc_gather_body(y_hbm, pos0_hbm, pos1_hbm, y0_hbm, y1_hbm, idx_v, r_v,
                    sem):
    wid = lax.axis_index("s") * SC_CORES + lax.axis_index("c")
    base = wid * TOK_CHUNK
    pltpu.sync_copy(pos0_hbm.at[pl.ds(base, TOK_CHUNK)], idx_v)
    pltpu.async_copy(y_hbm.at[idx_v], r_v, sem).wait()
    pltpu.sync_copy(r_v, y0_hbm.at[pl.ds(base, TOK_CHUNK)])
    pltpu.sync_copy(pos1_hbm.at[pl.ds(base, TOK_CHUNK)], idx_v)
    pltpu.async_copy(y_hbm.at[idx_v], r_v, sem).wait()
    pltpu.sync_copy(r_v, y1_hbm.at[pl.ds(base, TOK_CHUNK)])


def _sc_gather(y_sorted, pos0, pos1):
    mesh = plsc.VectorSubcoreMesh(core_axis_name="c", subcore_axis_name="s")
    f = pl.kernel(
        _sc_gather_body,
        out_type=[
            jax.ShapeDtypeStruct((SEQ, D_HALF), jnp.uint32),
            jax.ShapeDtypeStruct((SEQ, D_HALF), jnp.uint32),
        ],
        mesh=mesh,
        scratch_types=[
            pltpu.VMEM((TOK_CHUNK,), jnp.int32),
            pltpu.VMEM((TOK_CHUNK, D_HALF), jnp.uint32),
            pltpu.SemaphoreType.DMA,
        ],
    )
    return f(y_sorted, pos0, pos1)


def _combine_body(y0_ref, y1_ref, probs_ref, out_ref):
    p = probs_ref[...]
    y0 = _unpack_halves(y0_ref[...])
    y1 = _unpack_halves(y1_ref[...])
    out_ref[...] = p[:, 0:1] * y0 + p[:, 1:2] * y1


def _combine(y0, y1, probs):
    nblk = SEQ // TOK_BLK
    return pl.pallas_call(
        _combine_body,
        grid=(nblk,),
        in_specs=[
            pl.BlockSpec((TOK_BLK, D_HALF), lambda i: (i, 0)),
            pl.BlockSpec((TOK_BLK, D_HALF), lambda i: (i, 0)),
            pl.BlockSpec((TOK_BLK, TOP_K), lambda i: (i, 0)),
        ],
        out_specs=pl.BlockSpec((TOK_BLK, D_MODEL), lambda i: (i, 0)),
        out_shape=jax.ShapeDtypeStruct((SEQ, D_MODEL), jnp.float32),
    )(y0, y1, probs)


def kernel(x, Wg, sim_matrix, temperature, W1, b1, W2, b2):
    x2d = x.reshape(SEQ, D_MODEL)
    temp = temperature.reshape(1, 1)

    probs, eidx, rank, counts, xpk = _gate(x2d, Wg, sim_matrix, temp)
    counts = counts[0]

    nb = (counts + ROW_BLK - 1) // ROW_BLK
    er = jnp.arange(NUM_EXPERTS, dtype=jnp.int32)
    cum_nb = jnp.sum(jnp.where(er[:, None] >= er[None, :], nb[None, :], 0),
                     axis=1)
    first_blk = cum_nb - nb
    off_rows = first_blk * ROW_BLK
    erange = jnp.arange(NUM_EXPERTS, dtype=jnp.int32)
    pos = rank + jnp.sum(
        jnp.where(eidx[..., None] == erange, off_rows, 0), axis=-1)

    gids = jnp.arange(NUM_BLOCKS, dtype=jnp.int32)
    be_raw = jnp.sum(gids[:, None] >= cum_nb[None, :], axis=1)
    used = cum_nb[-1]
    last_e = jnp.clip(jnp.sum((used - 1) >= cum_nb), 0, NUM_EXPERTS - 1)
    block_expert = jnp.where(gids < used, jnp.clip(be_raw, 0, NUM_EXPERTS - 1),
                             last_e).astype(jnp.int32)

    pos0 = pos[:, 0]
    pos1 = pos[:, 1]
    x_sorted = _sc_scatter(xpk, pos0, pos1)

    prev_be = jnp.concatenate([block_expert[-1:], block_expert[:-1]])
    chg = ((gids < used) & ((gids == 0) | (block_expert != prev_be)))
    chg_i = chg.astype(jnp.int32)
    eord = jnp.sum(jnp.where(gids[:, None] >= gids[None, :],
                             chg_i[None, :], 0), axis=1) - 1
    slot_arr = jnp.where(gids < used, eord % 2, 0).astype(jnp.int32)
    cand = jnp.where(chg, gids, NUM_BLOCKS)
    later = jnp.where(gids[None, :] > gids[:, None], cand[None, :],
                      NUM_BLOCKS)
    su = jnp.min(later, axis=1)
    nxte_val = jnp.sum(
        jnp.where(su[:, None] == gids[None, :], block_expert[None, :], 0),
        axis=1)
    nxte = jnp.where(su < NUM_BLOCKS, nxte_val, -1).astype(jnp.int32)
    scal = jnp.concatenate(
        [block_expert, chg.astype(jnp.int32), slot_arr, nxte,
         used.astype(jnp.int32).reshape(1)])
    y_sorted = _grouped_mlp(x_sorted, scal, W1, b1, W2, b2)

    y0, y1 = _sc_gather(y_sorted, pos0, pos1)
    out = _combine(y0, y1, probs)
    return out.reshape(x.shape)
